# baseline (speedup 1.0000x reference)
import functools

import jax
import jax.numpy as jnp
from jax import lax
from jax.experimental import pallas as pl
from jax.experimental.pallas import tpu as pltpu
from jax.experimental.pallas import tpu_sc as plsc

VOCAB = 1000000
DIM = 64
BATCH = 4096
HIST = 200

NC, NS = 2, 16
NW = NC * NS
TOTAL = BATCH * HIST
B_PER_W = TOTAL // NW
SUB = 128
CHUNK = 512
SUBS_PER_CHUNK = CHUNK // SUB
N_CHUNKS = B_PER_W // CHUNK
IDX_ROWS = B_PER_W // SUB


def _gather_kernel(table_hbm, idx_hbm, out_hbm, idx_v, rows_v, sem):
    wid = lax.axis_index("s") * NC + lax.axis_index("c")
    base = wid * B_PER_W
    idx_row0 = wid * IDX_ROWS

    pltpu.sync_copy(idx_hbm.at[pl.ds(idx_row0, IDX_ROWS)], idx_v)

    def body(c, carry):
        for s in range(SUBS_PER_CHUNK):
            pltpu.async_copy(
                table_hbm.at[idx_v.at[c * SUBS_PER_CHUNK + s]],
                rows_v.at[pl.ds(s * SUB, SUB)],
                sem,
            )
        pltpu.make_async_copy(
            table_hbm.at[pl.ds(0, CHUNK)], rows_v, sem
        ).wait()
        pltpu.sync_copy(rows_v, out_hbm.at[pl.ds(base + c * CHUNK, CHUNK)])
        return carry

    lax.fori_loop(0, N_CHUNKS, body, 0)


@jax.jit
def _embedding_lookup(input_ids_2d, table):
    mesh = plsc.VectorSubcoreMesh(
        core_axis_name="c", subcore_axis_name="s",
        num_cores=NC, num_subcores=NS,
    )
    run = pl.kernel(
        _gather_kernel,
        out_type=jax.ShapeDtypeStruct((TOTAL, DIM), jnp.float32),
        mesh=mesh,
        scratch_types=[
            pltpu.VMEM((IDX_ROWS, SUB), jnp.int32),
            pltpu.VMEM((CHUNK, DIM), jnp.float32),
            pltpu.SemaphoreType.DMA,
        ],
        compiler_params=pltpu.CompilerParams(use_tc_tiling_on_sc=False),
    )
    return run(table, input_ids_2d)


def kernel(input_ids, table):
    idx2d = input_ids.reshape(TOTAL // SUB, SUB)
    out = _embedding_lookup(idx2d, table)
    return out.reshape(BATCH, HIST, DIM)


# speedup vs baseline: 1.0259x; 1.0259x over previous
import functools

import jax
import jax.numpy as jnp
from jax import lax
from jax.experimental import pallas as pl
from jax.experimental.pallas import tpu as pltpu
from jax.experimental.pallas import tpu_sc as plsc

VOCAB = 1000000
DIM = 64
BATCH = 4096
HIST = 200

NC, NS = 2, 16
NW = NC * NS
TOTAL = BATCH * HIST
B_PER_W = TOTAL // NW
SUB = 128
CHUNK = 256
SUBS_PER_CHUNK = CHUNK // SUB
N_CHUNKS = B_PER_W // CHUNK
IDX_ROWS = B_PER_W // SUB
NB = 4
N_OUTER = N_CHUNKS // NB


def _gather_kernel(table_hbm, idx_hbm, out_hbm, idx_v, rows, gsems, ssems):
    wid = lax.axis_index("s") * NC + lax.axis_index("c")
    base = wid * B_PER_W
    idx_row0 = wid * IDX_ROWS

    pltpu.sync_copy(idx_hbm.at[pl.ds(idx_row0, IDX_ROWS)], idx_v)

    def fire_gathers(k, b):
        for s in range(SUBS_PER_CHUNK):
            pltpu.async_copy(
                table_hbm.at[idx_v.at[k * SUBS_PER_CHUNK + s]],
                rows[b].at[pl.ds(s * SUB, SUB)],
                gsems[b],
            )

    def wait_gathers(b):
        pltpu.make_async_copy(
            table_hbm.at[pl.ds(0, CHUNK)], rows[b], gsems[b]
        ).wait()

    def start_store(k, b):
        pltpu.async_copy(
            rows[b], out_hbm.at[pl.ds(base + k * CHUNK, CHUNK)], ssems[b]
        )

    def wait_store(b):
        pltpu.make_async_copy(
            rows[b], out_hbm.at[pl.ds(base, CHUNK)], ssems[b]
        ).wait()

    fire_gathers(0, 0)
    fire_gathers(1, 1)

    def body(c, carry):
        for b in range(NB):
            p = c * NB + b
            b2 = (b + 2) % NB
            if b < 2:
                @pl.when(c > 0)
                def _():
                    wait_store(b2)
                fire_gathers(p + 2, b2)
            else:
                @pl.when(c < N_OUTER - 1)
                def _():
                    wait_store(b2)
                    fire_gathers(p + 2, b2)
            wait_gathers(b)
            start_store(p, b)
        return carry

    lax.fori_loop(0, N_OUTER, body, 0)

    for b in range(NB):
        wait_store(b)


@jax.jit
def _embedding_lookup(input_ids_2d, table):
    mesh = plsc.VectorSubcoreMesh(
        core_axis_name="c", subcore_axis_name="s",
        num_cores=NC, num_subcores=NS,
    )
    run = pl.kernel(
        _gather_kernel,
        out_type=jax.ShapeDtypeStruct((TOTAL, DIM), jnp.float32),
        mesh=mesh,
        scratch_types=[
            pltpu.VMEM((IDX_ROWS, SUB), jnp.int32),
            [pltpu.VMEM((CHUNK, DIM), jnp.float32) for _ in range(NB)],
            [pltpu.SemaphoreType.DMA for _ in range(NB)],
            [pltpu.SemaphoreType.DMA for _ in range(NB)],
        ],
        compiler_params=pltpu.CompilerParams(use_tc_tiling_on_sc=False),
    )
    return run(table, input_ids_2d)


def kernel(input_ids, table):
    idx2d = input_ids.reshape(TOTAL // SUB, SUB)
    out = _embedding_lookup(idx2d, table)
    return out.reshape(BATCH, HIST, DIM)


# speedup vs baseline: 1.0559x; 1.0293x over previous
import functools

import jax
import jax.numpy as jnp
from jax import lax
from jax.experimental import pallas as pl
from jax.experimental.pallas import tpu as pltpu
from jax.experimental.pallas import tpu_sc as plsc

VOCAB = 1000000
DIM = 64
BATCH = 4096
HIST = 200

NC, NS = 2, 16
NW = NC * NS
BC = BATCH // NW
N_UNITS = HIST
NB = 8
AHEAD = 4
N_OUTER = N_UNITS // NB


def _gather_kernel(table_hbm, ids_hbm, out_hbm, idx_v, bufs, gsems, ssems):
    wid = lax.axis_index("s") * NC + lax.axis_index("c")
    b0 = wid * BC

    pltpu.sync_copy(ids_hbm.at[:, pl.ds(b0, BC)], idx_v)

    def fire_gather(h, g):
        pltpu.async_copy(table_hbm.at[idx_v.at[h]], bufs[g], gsems[g])

    def wait_gather(g):
        pltpu.make_async_copy(
            table_hbm.at[pl.ds(0, BC)], bufs[g], gsems[g]
        ).wait()

    def start_store(h, g):
        pltpu.async_copy(
            bufs[g], out_hbm.at[h, pl.ds(b0, BC), :], ssems[g]
        )

    def wait_store(g):
        pltpu.make_async_copy(
            bufs[g], out_hbm.at[0, pl.ds(b0, BC), :], ssems[g]
        ).wait()

    for g in range(AHEAD):
        fire_gather(g, g)

    def body(c, carry):
        for u in range(NB):
            p = c * NB + u
            s = (u + AHEAD) % NB
            if u < AHEAD:
                @pl.when(c > 0)
                def _():
                    wait_store(s)
                fire_gather(p + AHEAD, s)
            else:
                @pl.when(c < N_OUTER - 1)
                def _():
                    wait_store(s)
                    fire_gather(p + AHEAD, s)
            wait_gather(u)
            start_store(p, u)
        return carry

    lax.fori_loop(0, N_OUTER, body, 0)

    for g in range(NB):
        wait_store(g)


@jax.jit
def _embedding_lookup(ids_t, table):
    mesh = plsc.VectorSubcoreMesh(
        core_axis_name="c", subcore_axis_name="s",
        num_cores=NC, num_subcores=NS,
    )
    run = pl.kernel(
        _gather_kernel,
        out_type=jax.ShapeDtypeStruct((HIST, BATCH, DIM), jnp.float32),
        mesh=mesh,
        scratch_types=[
            pltpu.VMEM((HIST, BC), jnp.int32),
            [pltpu.VMEM((BC, DIM), jnp.float32) for _ in range(NB)],
            [pltpu.SemaphoreType.DMA for _ in range(NB)],
            [pltpu.SemaphoreType.DMA for _ in range(NB)],
        ],
        compiler_params=pltpu.CompilerParams(
            use_tc_tiling_on_sc=False, needs_layout_passes=False,
        ),
    )
    return run(table, ids_t)


def kernel(input_ids, table):
    ids_t = input_ids.T
    out_t = _embedding_lookup(ids_t, table)
    return jnp.transpose(out_t, (1, 0, 2))
